# baseline (speedup 1.0000x reference)
import functools

import jax
import jax.numpy as jnp
from jax import lax
from jax.experimental import pallas as pl
from jax.experimental.pallas import tpu as pltpu
from jax.experimental.pallas import tpu_sc as plsc

N, E, F = 10000, 320000, 128
NC, NS = 2, 16
NW = NC * NS
L = 16
NSL = F // L

EW = E // NW
CB = 80
NCH = EW // CB
NPAIR = (NCH + 1) // 2

PT = 320
NLAST = N - (NW - 1) * PT
DC = 4000
NDC = E // DC
GRP = 1
MB_CAP = 512
FLUSH_AT = MB_CAP - GRP * L

BN = 1000
BE = 2560


def _uv_body(x_ref, w_ref, b_ref, u_ref, v_ref):
    uv = jnp.dot(x_ref[...], w_ref[...], preferred_element_type=jnp.float32)
    uv = uv + b_ref[...]
    u_ref[...] = uv[:, :F]
    v_ref[...] = uv[:, F:]


def _mlp_body(xl_ref, g_ref, w1t_ref, w2t_ref, b2_ref, m_ref):
    l1 = jnp.dot(
        xl_ref[...].astype(jnp.bfloat16),
        w1t_ref[...].astype(jnp.bfloat16),
        preferred_element_type=jnp.float32,
    )
    h = jnp.maximum(g_ref[...] + l1, 0.0)
    m_ref[...] = (
        jnp.dot(
            h.astype(jnp.bfloat16),
            w2t_ref[...].astype(jnp.bfloat16),
            preferred_element_type=jnp.float32,
        )
        + b2_ref[...]
    )


def _gather_body(u_hbm, v_hbm, dst_hbm, src_hbm, g_hbm,
                 didx, sidx, ua, va, ub, vb, usemA, vsemA, usemB, vsemB):
    wid = lax.axis_index("s") * NC + lax.axis_index("c")
    base = wid * EW
    pltpu.sync_copy(dst_hbm.at[pl.ds(base, EW)], didx)
    pltpu.sync_copy(src_hbm.at[pl.ds(base, EW)], sidx)

    def fire(c, urows, vrows, us, vs):
        off = c * CB
        pltpu.async_copy(u_hbm.at[didx.at[pl.ds(off, CB)]], urows, us)
        pltpu.async_copy(v_hbm.at[sidx.at[pl.ds(off, CB)]], vrows, vs)

    def drain(urows, vrows, us, vs):
        pltpu.make_async_copy(u_hbm.at[didx.at[pl.ds(0, CB)]], urows, us).wait()
        pltpu.make_async_copy(v_hbm.at[sidx.at[pl.ds(0, CB)]], vrows, vs).wait()

    def add_and_store(c, urows, vrows):
        def addrow(r, carry):
            for rr in (2 * r, 2 * r + 1):
                for j in range(NSL):
                    sl = pl.ds(j * L, L)
                    plsc.addupdate(urows.at[rr, sl], vrows[rr, sl])
            return carry

        lax.fori_loop(0, CB // 2, addrow, 0)
        pltpu.sync_copy(urows, g_hbm.at[pl.ds(base + c * CB, CB)])

    fire(0, ua, va, usemA, vsemA)

    def pair(p, carry):
        c0 = 2 * p
        c1 = jnp.minimum(2 * p + 1, NCH - 1)
        fire(c1, ub, vb, usemB, vsemB)
        drain(ua, va, usemA, vsemA)
        add_and_store(c0, ua, va)
        fire(jnp.minimum(c0 + 2, NCH - 1), ua, va, usemA, vsemA)
        drain(ub, vb, usemB, vsemB)
        add_and_store(c1, ub, vb)
        return carry

    lax.fori_loop(0, NPAIR, pair, 0)
    drain(ua, va, usemA, vsemA)


def _segmax_body(dst_hbm, m_hbm, out_hbm, dca, dcb, mid, mld, rows, acc,
                 gsem, dsemA, dsemB):
    wid = lax.axis_index("s") * NC + lax.axis_index("c")
    lo = wid * PT
    neg_inf = jnp.full((L,), -jnp.inf, dtype=jnp.float32)

    def init_acc(i, carry):
        for j in range(NSL):
            acc[i, pl.ds(j * L, L)] = neg_inf
        return carry

    lax.fori_loop(0, PT, init_acc, 0)
    zeros_i = jnp.zeros((L,), dtype=jnp.int32)
    for b in range(MB_CAP // L):
        mid[pl.ds(b * L, L)] = zeros_i
    for b in range(MB_CAP // L + 1):
        mld[pl.ds(b * L, L)] = zeros_i

    def flush(nm):
        for q in range(MB_CAP // 128):
            pltpu.async_copy(
                m_hbm.at[mid.at[pl.ds(q * 128, 128)]],
                rows.at[pl.ds(q * 128, 128)], gsem)
        for q in range(MB_CAP // 128):
            pltpu.make_async_copy(
                m_hbm.at[mid.at[pl.ds(q * 128, 128)]],
                rows.at[pl.ds(q * 128, 128)], gsem).wait()

        def apply1(k, carry):
            ld = mld[pl.ds(k, L)][0]
            for j in range(NSL):
                sl = pl.ds(j * L, L)
                acc[ld, sl] = jnp.maximum(acc[ld, sl], rows[k, sl])
            return carry

        lax.fori_loop(0, nm, apply1, 0)
        return jnp.int32(0)

    iota = lax.iota(jnp.int32, L)

    def scan_chunk(dstc, c, nm):
        def group(g, nm):
            for u in range(GRP):
                s = g * GRP + u
                d = dstc[pl.ds(s * L, L)]
                ldv = d - lo
                msk = (ldv >= 0) & (ldv < PT)
                cnt = plsc.all_reduce_population_count(msk)[0]
                eids = (c * DC + s * L) + iota
                plsc.store_compressed(mid.at[pl.ds(nm, L)], eids, mask=msk)
                plsc.store_compressed(mld.at[pl.ds(nm, L)], ldv, mask=msk)
                nm = nm + cnt
            return lax.cond(nm >= FLUSH_AT, flush, lambda x: x, nm)

        return lax.fori_loop(0, DC // L // GRP, group, nm)

    def dfire(c, buf, sem):
        pltpu.async_copy(dst_hbm.at[pl.ds(c * DC, DC)], buf, sem)

    def ddrain(buf, sem):
        pltpu.make_async_copy(dst_hbm.at[pl.ds(0, DC)], buf, sem).wait()

    def dfire(c, buf, sem):
        pltpu.async_copy(dst_hbm.at[pl.ds(c * DC, DC)], buf, sem)

    def ddrain(buf, sem):
        pltpu.make_async_copy(dst_hbm.at[pl.ds(0, DC)], buf, sem).wait()

    dfire(0, dca, dsemA)

    def pair(p, nm):
        c0 = 2 * p
        c1 = 2 * p + 1
        dfire(c1, dcb, dsemB)
        ddrain(dca, dsemA)
        nm = scan_chunk(dca, c0, nm)
        dfire(jnp.minimum(c0 + 2, NDC - 1), dca, dsemA)
        ddrain(dcb, dsemB)
        nm = scan_chunk(dcb, c1, nm)
        return nm

    nm = lax.fori_loop(0, NDC // 2, pair, jnp.int32(0))
    ddrain(dca, dsemA)
    flush(nm)

    def finish(i, carry):
        for j in range(NSL):
            sl = pl.ds(j * L, L)
            v = acc[i, sl]
            acc[i, sl] = jnp.where(v == -jnp.inf, 0.0, v)
        return carry

    lax.fori_loop(0, PT, finish, 0)

    @pl.when(wid < NW - 1)
    def _():
        pltpu.sync_copy(acc, out_hbm.at[pl.ds(lo, PT)])

    @pl.when(wid == NW - 1)
    def _():
        pltpu.sync_copy(acc.at[pl.ds(0, NLAST)], out_hbm.at[pl.ds(lo, NLAST)])


_sc_mesh = plsc.VectorSubcoreMesh(
    core_axis_name="c", subcore_axis_name="s", num_cores=NC, num_subcores=NS
)

_sc_params = pltpu.CompilerParams(needs_layout_passes=False)

_gather_kernel = functools.partial(
    pl.kernel,
    mesh=_sc_mesh,
    compiler_params=_sc_params,
    out_type=jax.ShapeDtypeStruct((E, F), jnp.float32),
    scratch_types=[
        pltpu.VMEM((EW,), jnp.int32),
        pltpu.VMEM((EW,), jnp.int32),
        pltpu.VMEM((CB, F), jnp.float32),
        pltpu.VMEM((CB, F), jnp.float32),
        pltpu.VMEM((CB, F), jnp.float32),
        pltpu.VMEM((CB, F), jnp.float32),
        pltpu.SemaphoreType.DMA,
        pltpu.SemaphoreType.DMA,
        pltpu.SemaphoreType.DMA,
        pltpu.SemaphoreType.DMA,
    ],
)(_gather_body)

_segmax_kernel = functools.partial(
    pl.kernel,
    mesh=_sc_mesh,
    compiler_params=_sc_params,
    out_type=jax.ShapeDtypeStruct((N, F), jnp.float32),
    scratch_types=[
        pltpu.VMEM((DC,), jnp.int32),
        pltpu.VMEM((DC,), jnp.int32),
        pltpu.VMEM((MB_CAP,), jnp.int32),
        pltpu.VMEM((MB_CAP + L,), jnp.int32),
        pltpu.VMEM((MB_CAP, F), jnp.float32),
        pltpu.VMEM((PT, F), jnp.float32),
        pltpu.SemaphoreType.DMA,
        pltpu.SemaphoreType.DMA,
        pltpu.SemaphoreType.DMA,
    ],
)(_segmax_body)


def kernel(x, edge_index, x_label, W_label, W1, b1, W2, b2):
    src = edge_index[0]
    dst = edge_index[1]
    P = W_label[:, :F].T
    Q = W_label[:, F:].T
    w_uv = jnp.concatenate([(P - Q) @ W1.T, Q @ W1.T], axis=1)
    b_uv = jnp.concatenate([b1, jnp.zeros((F,), jnp.float32)])[None, :]

    u, v = pl.pallas_call(
        _uv_body,
        grid=(N // BN,),
        in_specs=[
            pl.BlockSpec((BN, F), lambda i: (i, 0)),
            pl.BlockSpec((F, 2 * F), lambda i: (0, 0)),
            pl.BlockSpec((1, 2 * F), lambda i: (0, 0)),
        ],
        out_specs=[
            pl.BlockSpec((BN, F), lambda i: (i, 0)),
            pl.BlockSpec((BN, F), lambda i: (i, 0)),
        ],
        out_shape=[
            jax.ShapeDtypeStruct((N, F), jnp.float32),
            jax.ShapeDtypeStruct((N, F), jnp.float32),
        ],
    )(x, w_uv, b_uv)

    g = _gather_kernel(u, v, dst, src)

    m = pl.pallas_call(
        _mlp_body,
        grid=(E // BE,),
        in_specs=[
            pl.BlockSpec((BE, F), lambda i: (i, 0)),
            pl.BlockSpec((BE, F), lambda i: (i, 0)),
            pl.BlockSpec((F, F), lambda i: (0, 0)),
            pl.BlockSpec((F, F), lambda i: (0, 0)),
            pl.BlockSpec((1, F), lambda i: (0, 0)),
        ],
        out_specs=pl.BlockSpec((BE, F), lambda i: (i, 0)),
        out_shape=jax.ShapeDtypeStruct((E, F), jnp.float32),
    )(x_label, g, W1.T, W2.T, b2[None, :])

    return _segmax_kernel(dst, m)


# speedup vs baseline: 1.0219x; 1.0219x over previous
import functools

import jax
import jax.numpy as jnp
from jax import lax
from jax.experimental import pallas as pl
from jax.experimental.pallas import tpu as pltpu
from jax.experimental.pallas import tpu_sc as plsc

N, E, F = 10000, 320000, 128
NC, NS = 2, 16
NW = NC * NS
L = 16
NSL = F // L

EW = E // NW
CB = 80
NCH = EW // CB
NPAIR = (NCH + 1) // 2

PT = 320
NLAST = N - (NW - 1) * PT
DC = 4000
NDC = E // DC
GRP = 1
MB_CAP = 512
FLUSH_AT = MB_CAP - GRP * L

BN = 1000
BE = 2560


def _uv_body(x_ref, w_ref, b_ref, u_ref, v_ref):
    uv = jnp.dot(x_ref[...], w_ref[...], preferred_element_type=jnp.float32)
    uv = uv + b_ref[...]
    u_ref[...] = uv[:, :F]
    v_ref[...] = uv[:, F:]


def _mlp_body(xl_ref, g_ref, w1t_ref, w2t_ref, b2_ref, m_ref):
    l1 = jnp.dot(
        xl_ref[...].astype(jnp.bfloat16),
        w1t_ref[...].astype(jnp.bfloat16),
        preferred_element_type=jnp.float32,
    )
    h = jnp.maximum(g_ref[...] + l1, 0.0)
    m_ref[...] = (
        jnp.dot(
            h.astype(jnp.bfloat16),
            w2t_ref[...].astype(jnp.bfloat16),
            preferred_element_type=jnp.float32,
        )
        + b2_ref[...]
    )


def _gather_body(u_hbm, v_hbm, dst_hbm, src_hbm, g_hbm,
                 didx, sidx, ua, va, ub, vb, usemA, vsemA, usemB, vsemB):
    wid = lax.axis_index("s") * NC + lax.axis_index("c")
    base = wid * EW
    pltpu.sync_copy(dst_hbm.at[pl.ds(base, EW)], didx)
    pltpu.sync_copy(src_hbm.at[pl.ds(base, EW)], sidx)

    def fire(c, urows, vrows, us, vs):
        off = c * CB
        pltpu.async_copy(u_hbm.at[didx.at[pl.ds(off, CB)]], urows, us)
        pltpu.async_copy(v_hbm.at[sidx.at[pl.ds(off, CB)]], vrows, vs)

    def drain(urows, vrows, us, vs):
        pltpu.make_async_copy(u_hbm.at[didx.at[pl.ds(0, CB)]], urows, us).wait()
        pltpu.make_async_copy(v_hbm.at[sidx.at[pl.ds(0, CB)]], vrows, vs).wait()

    def add_and_store(c, urows, vrows):
        def addrow(r, carry):
            for rr in (2 * r, 2 * r + 1):
                for j in range(NSL):
                    sl = pl.ds(j * L, L)
                    plsc.addupdate(urows.at[rr, sl], vrows[rr, sl])
            return carry

        lax.fori_loop(0, CB // 2, addrow, 0)
        pltpu.sync_copy(urows, g_hbm.at[pl.ds(base + c * CB, CB)])

    fire(0, ua, va, usemA, vsemA)

    def pair(p, carry):
        c0 = 2 * p
        c1 = jnp.minimum(2 * p + 1, NCH - 1)
        fire(c1, ub, vb, usemB, vsemB)
        drain(ua, va, usemA, vsemA)
        add_and_store(c0, ua, va)
        fire(jnp.minimum(c0 + 2, NCH - 1), ua, va, usemA, vsemA)
        drain(ub, vb, usemB, vsemB)
        add_and_store(c1, ub, vb)
        return carry

    lax.fori_loop(0, NPAIR, pair, 0)
    drain(ua, va, usemA, vsemA)


def _segmax_body(dst_hbm, m_hbm, out_hbm, dca, dcb, mid, mld, rows, acc,
                 gsems, dsemA, dsemB):
    wid = lax.axis_index("s") * NC + lax.axis_index("c")
    lo = wid * PT
    neg_inf = jnp.full((L,), -jnp.inf, dtype=jnp.float32)

    def init_acc(i, carry):
        for j in range(NSL):
            acc[i, pl.ds(j * L, L)] = neg_inf
        return carry

    lax.fori_loop(0, PT, init_acc, 0)
    zeros_i = jnp.zeros((L,), dtype=jnp.int32)
    for b in range(MB_CAP // L):
        mid[pl.ds(b * L, L)] = zeros_i
    for b in range(MB_CAP // L + 1):
        mld[pl.ds(b * L, L)] = zeros_i

    def flush(nm):
        for q in range(MB_CAP // 128):
            pltpu.async_copy(
                m_hbm.at[mid.at[pl.ds(q * 128, 128)]],
                rows.at[pl.ds(q * 128, 128)], gsems.at[q])

        def apply1(k, carry):
            ld = mld[pl.ds(k, L)][0]
            for j in range(NSL):
                sl = pl.ds(j * L, L)
                acc[ld, sl] = jnp.maximum(acc[ld, sl], rows[k, sl])
            return carry

        for q in range(MB_CAP // 128):
            pltpu.make_async_copy(
                m_hbm.at[mid.at[pl.ds(q * 128, 128)]],
                rows.at[pl.ds(q * 128, 128)], gsems.at[q]).wait()
            hi = jnp.minimum(nm, (q + 1) * 128)
            lax.fori_loop(jnp.minimum(hi, q * 128), hi, apply1, 0)
        return jnp.int32(0)

    iota = lax.iota(jnp.int32, L)

    def scan_chunk(dstc, c, nm):
        def group(g, nm):
            for u in range(GRP):
                s = g * GRP + u
                d = dstc[pl.ds(s * L, L)]
                ldv = d - lo
                msk = (ldv >= 0) & (ldv < PT)
                cnt = plsc.all_reduce_population_count(msk)[0]
                eids = (c * DC + s * L) + iota
                plsc.store_compressed(mid.at[pl.ds(nm, L)], eids, mask=msk)
                plsc.store_compressed(mld.at[pl.ds(nm, L)], ldv, mask=msk)
                nm = nm + cnt
            return lax.cond(nm >= FLUSH_AT, flush, lambda x: x, nm)

        return lax.fori_loop(0, DC // L // GRP, group, nm)

    def dfire(c, buf, sem):
        pltpu.async_copy(dst_hbm.at[pl.ds(c * DC, DC)], buf, sem)

    def ddrain(buf, sem):
        pltpu.make_async_copy(dst_hbm.at[pl.ds(0, DC)], buf, sem).wait()

    def dfire(c, buf, sem):
        pltpu.async_copy(dst_hbm.at[pl.ds(c * DC, DC)], buf, sem)

    def ddrain(buf, sem):
        pltpu.make_async_copy(dst_hbm.at[pl.ds(0, DC)], buf, sem).wait()

    dfire(0, dca, dsemA)

    def pair(p, nm):
        c0 = 2 * p
        c1 = 2 * p + 1
        dfire(c1, dcb, dsemB)
        ddrain(dca, dsemA)
        nm = scan_chunk(dca, c0, nm)
        dfire(jnp.minimum(c0 + 2, NDC - 1), dca, dsemA)
        ddrain(dcb, dsemB)
        nm = scan_chunk(dcb, c1, nm)
        return nm

    nm = lax.fori_loop(0, NDC // 2, pair, jnp.int32(0))
    ddrain(dca, dsemA)
    flush(nm)

    def finish(i, carry):
        for j in range(NSL):
            sl = pl.ds(j * L, L)
            v = acc[i, sl]
            acc[i, sl] = jnp.where(v == -jnp.inf, 0.0, v)
        return carry

    lax.fori_loop(0, PT, finish, 0)

    @pl.when(wid < NW - 1)
    def _():
        pltpu.sync_copy(acc, out_hbm.at[pl.ds(lo, PT)])

    @pl.when(wid == NW - 1)
    def _():
        pltpu.sync_copy(acc.at[pl.ds(0, NLAST)], out_hbm.at[pl.ds(lo, NLAST)])


_sc_mesh = plsc.VectorSubcoreMesh(
    core_axis_name="c", subcore_axis_name="s", num_cores=NC, num_subcores=NS
)

_sc_params = pltpu.CompilerParams(needs_layout_passes=False)

_gather_kernel = functools.partial(
    pl.kernel,
    mesh=_sc_mesh,
    compiler_params=_sc_params,
    out_type=jax.ShapeDtypeStruct((E, F), jnp.float32),
    scratch_types=[
        pltpu.VMEM((EW,), jnp.int32),
        pltpu.VMEM((EW,), jnp.int32),
        pltpu.VMEM((CB, F), jnp.float32),
        pltpu.VMEM((CB, F), jnp.float32),
        pltpu.VMEM((CB, F), jnp.float32),
        pltpu.VMEM((CB, F), jnp.float32),
        pltpu.SemaphoreType.DMA,
        pltpu.SemaphoreType.DMA,
        pltpu.SemaphoreType.DMA,
        pltpu.SemaphoreType.DMA,
    ],
)(_gather_body)

_segmax_kernel = functools.partial(
    pl.kernel,
    mesh=_sc_mesh,
    compiler_params=_sc_params,
    out_type=jax.ShapeDtypeStruct((N, F), jnp.float32),
    scratch_types=[
        pltpu.VMEM((DC,), jnp.int32),
        pltpu.VMEM((DC,), jnp.int32),
        pltpu.VMEM((MB_CAP,), jnp.int32),
        pltpu.VMEM((MB_CAP + L,), jnp.int32),
        pltpu.VMEM((MB_CAP, F), jnp.float32),
        pltpu.VMEM((PT, F), jnp.float32),
        pltpu.SemaphoreType.DMA((MB_CAP // 128,)),
        pltpu.SemaphoreType.DMA,
        pltpu.SemaphoreType.DMA,
    ],
)(_segmax_body)


def kernel(x, edge_index, x_label, W_label, W1, b1, W2, b2):
    src = edge_index[0]
    dst = edge_index[1]
    P = W_label[:, :F].T
    Q = W_label[:, F:].T
    w_uv = jnp.concatenate([(P - Q) @ W1.T, Q @ W1.T], axis=1)
    b_uv = jnp.concatenate([b1, jnp.zeros((F,), jnp.float32)])[None, :]

    u, v = pl.pallas_call(
        _uv_body,
        grid=(N // BN,),
        in_specs=[
            pl.BlockSpec((BN, F), lambda i: (i, 0)),
            pl.BlockSpec((F, 2 * F), lambda i: (0, 0)),
            pl.BlockSpec((1, 2 * F), lambda i: (0, 0)),
        ],
        out_specs=[
            pl.BlockSpec((BN, F), lambda i: (i, 0)),
            pl.BlockSpec((BN, F), lambda i: (i, 0)),
        ],
        out_shape=[
            jax.ShapeDtypeStruct((N, F), jnp.float32),
            jax.ShapeDtypeStruct((N, F), jnp.float32),
        ],
    )(x, w_uv, b_uv)

    g = _gather_kernel(u, v, dst, src)

    m = pl.pallas_call(
        _mlp_body,
        grid=(E // BE,),
        in_specs=[
            pl.BlockSpec((BE, F), lambda i: (i, 0)),
            pl.BlockSpec((BE, F), lambda i: (i, 0)),
            pl.BlockSpec((F, F), lambda i: (0, 0)),
            pl.BlockSpec((F, F), lambda i: (0, 0)),
            pl.BlockSpec((1, F), lambda i: (0, 0)),
        ],
        out_specs=pl.BlockSpec((BE, F), lambda i: (i, 0)),
        out_shape=jax.ShapeDtypeStruct((E, F), jnp.float32),
    )(x_label, g, W1.T, W2.T, b2[None, :])

    return _segmax_kernel(dst, m)


# speedup vs baseline: 1.0231x; 1.0011x over previous
import functools

import jax
import jax.numpy as jnp
from jax import lax
from jax.experimental import pallas as pl
from jax.experimental.pallas import tpu as pltpu
from jax.experimental.pallas import tpu_sc as plsc

N, E, F = 10000, 320000, 128
NC, NS = 2, 16
NW = NC * NS
L = 16
NSL = F // L

EW = E // NW
CB = 80
NCH = EW // CB
NPAIR = (NCH + 1) // 2

PT = 320
NLAST = N - (NW - 1) * PT
DC = 4000
NDC = E // DC
GRP = 1
MB_CAP = 512
FLUSH_AT = MB_CAP - GRP * L

BN = 1000
BE = 2560


def _uv_body(x_ref, w_ref, b_ref, u_ref, v_ref):
    uv = jnp.dot(x_ref[...], w_ref[...], preferred_element_type=jnp.float32)
    uv = uv + b_ref[...]
    u_ref[...] = uv[:, :F]
    v_ref[...] = uv[:, F:]


def _mlp_body(xl_ref, g_ref, w1t_ref, w2t_ref, b2_ref, m_ref):
    l1 = jnp.dot(
        xl_ref[...].astype(jnp.bfloat16),
        w1t_ref[...].astype(jnp.bfloat16),
        preferred_element_type=jnp.float32,
    )
    h = jnp.maximum(g_ref[...] + l1, 0.0)
    m_ref[...] = (
        jnp.dot(
            h.astype(jnp.bfloat16),
            w2t_ref[...].astype(jnp.bfloat16),
            preferred_element_type=jnp.float32,
        )
        + b2_ref[...]
    )


def _gather_body(u_hbm, v_hbm, dst_hbm, src_hbm, g_hbm,
                 didx, sidx, ua, va, ub, vb, usemA, vsemA, usemB, vsemB):
    wid = lax.axis_index("s") * NC + lax.axis_index("c")
    base = wid * EW
    pltpu.sync_copy(dst_hbm.at[pl.ds(base, EW)], didx)
    pltpu.sync_copy(src_hbm.at[pl.ds(base, EW)], sidx)

    def fire(c, urows, vrows, us, vs):
        off = c * CB
        pltpu.async_copy(u_hbm.at[didx.at[pl.ds(off, CB)]], urows, us)
        pltpu.async_copy(v_hbm.at[sidx.at[pl.ds(off, CB)]], vrows, vs)

    def drain(urows, vrows, us, vs):
        pltpu.make_async_copy(u_hbm.at[didx.at[pl.ds(0, CB)]], urows, us).wait()
        pltpu.make_async_copy(v_hbm.at[sidx.at[pl.ds(0, CB)]], vrows, vs).wait()

    def add_and_store(c, urows, vrows):
        def addrow(r, carry):
            for rr in (2 * r, 2 * r + 1):
                for j in range(NSL):
                    sl = pl.ds(j * L, L)
                    plsc.addupdate(urows.at[rr, sl], vrows[rr, sl])
            return carry

        lax.fori_loop(0, CB // 2, addrow, 0)
        pltpu.sync_copy(urows, g_hbm.at[pl.ds(base + c * CB, CB)])

    fire(0, ua, va, usemA, vsemA)

    def pair(p, carry):
        c0 = 2 * p
        c1 = jnp.minimum(2 * p + 1, NCH - 1)
        fire(c1, ub, vb, usemB, vsemB)
        drain(ua, va, usemA, vsemA)
        add_and_store(c0, ua, va)
        fire(jnp.minimum(c0 + 2, NCH - 1), ua, va, usemA, vsemA)
        drain(ub, vb, usemB, vsemB)
        add_and_store(c1, ub, vb)
        return carry

    lax.fori_loop(0, NPAIR, pair, 0)
    drain(ua, va, usemA, vsemA)


def _segmax_body(dst_hbm, m_hbm, out_hbm, dca, dcb, mid, mld, rows, acc,
                 gsems, dsemA, dsemB):
    wid = lax.axis_index("s") * NC + lax.axis_index("c")
    lo = wid * PT
    neg_inf = jnp.full((L,), -jnp.inf, dtype=jnp.float32)

    def init_acc(i, carry):
        for j in range(NSL):
            acc[i, pl.ds(j * L, L)] = neg_inf
        return carry

    lax.fori_loop(0, PT, init_acc, 0)
    zeros_i = jnp.zeros((L,), dtype=jnp.int32)
    for b in range(MB_CAP // L):
        mid[pl.ds(b * L, L)] = zeros_i
    for b in range(MB_CAP // L + 1):
        mld[pl.ds(b * L, L)] = zeros_i

    def flush(nm):
        for q in range(MB_CAP // 128):
            pltpu.async_copy(
                m_hbm.at[mid.at[pl.ds(q * 128, 128)]],
                rows.at[pl.ds(q * 128, 128)], gsems.at[q])

        def apply1(k, carry):
            ld = mld[pl.ds(k, L)][0]
            for j in range(NSL):
                sl = pl.ds(j * L, L)
                acc[ld, sl] = jnp.maximum(acc[ld, sl], rows[k, sl])
            return carry

        for q in range(MB_CAP // 128):
            pltpu.make_async_copy(
                m_hbm.at[mid.at[pl.ds(q * 128, 128)]],
                rows.at[pl.ds(q * 128, 128)], gsems.at[q]).wait()
            hi = jnp.minimum(nm, (q + 1) * 128)
            lax.fori_loop(jnp.minimum(hi, q * 128), hi, apply1, 0)
        return jnp.int32(0)

    iota = lax.iota(jnp.int32, L)

    def scan_chunk(dstc, c, nm):
        def group(g, nm):
            for u in range(GRP):
                s = g * GRP + u
                d = dstc[pl.ds(s * L, L)]
                ldv = d - lo
                msk = plsc.bitcast(ldv, jnp.uint32) < jnp.uint32(PT)
                cnt = plsc.all_reduce_population_count(msk)[0]
                eids = (c * DC + s * L) + iota
                plsc.store_compressed(mid.at[pl.ds(nm, L)], eids, mask=msk)
                plsc.store_compressed(mld.at[pl.ds(nm, L)], ldv, mask=msk)
                nm = nm + cnt
            return lax.cond(nm >= FLUSH_AT, flush, lambda x: x, nm)

        return lax.fori_loop(0, DC // L // GRP, group, nm)

    def dfire(c, buf, sem):
        pltpu.async_copy(dst_hbm.at[pl.ds(c * DC, DC)], buf, sem)

    def ddrain(buf, sem):
        pltpu.make_async_copy(dst_hbm.at[pl.ds(0, DC)], buf, sem).wait()

    def dfire(c, buf, sem):
        pltpu.async_copy(dst_hbm.at[pl.ds(c * DC, DC)], buf, sem)

    def ddrain(buf, sem):
        pltpu.make_async_copy(dst_hbm.at[pl.ds(0, DC)], buf, sem).wait()

    dfire(0, dca, dsemA)

    def pair(p, nm):
        c0 = 2 * p
        c1 = 2 * p + 1
        dfire(c1, dcb, dsemB)
        ddrain(dca, dsemA)
        nm = scan_chunk(dca, c0, nm)
        dfire(jnp.minimum(c0 + 2, NDC - 1), dca, dsemA)
        ddrain(dcb, dsemB)
        nm = scan_chunk(dcb, c1, nm)
        return nm

    nm = lax.fori_loop(0, NDC // 2, pair, jnp.int32(0))
    ddrain(dca, dsemA)
    flush(nm)

    def finish(i, carry):
        for j in range(NSL):
            sl = pl.ds(j * L, L)
            v = acc[i, sl]
            acc[i, sl] = jnp.where(v == -jnp.inf, 0.0, v)
        return carry

    lax.fori_loop(0, PT, finish, 0)

    @pl.when(wid < NW - 1)
    def _():
        pltpu.sync_copy(acc, out_hbm.at[pl.ds(lo, PT)])

    @pl.when(wid == NW - 1)
    def _():
        pltpu.sync_copy(acc.at[pl.ds(0, NLAST)], out_hbm.at[pl.ds(lo, NLAST)])


_sc_mesh = plsc.VectorSubcoreMesh(
    core_axis_name="c", subcore_axis_name="s", num_cores=NC, num_subcores=NS
)

_sc_params = pltpu.CompilerParams(needs_layout_passes=False)

_gather_kernel = functools.partial(
    pl.kernel,
    mesh=_sc_mesh,
    compiler_params=_sc_params,
    out_type=jax.ShapeDtypeStruct((E, F), jnp.float32),
    scratch_types=[
        pltpu.VMEM((EW,), jnp.int32),
        pltpu.VMEM((EW,), jnp.int32),
        pltpu.VMEM((CB, F), jnp.float32),
        pltpu.VMEM((CB, F), jnp.float32),
        pltpu.VMEM((CB, F), jnp.float32),
        pltpu.VMEM((CB, F), jnp.float32),
        pltpu.SemaphoreType.DMA,
        pltpu.SemaphoreType.DMA,
        pltpu.SemaphoreType.DMA,
        pltpu.SemaphoreType.DMA,
    ],
)(_gather_body)

_segmax_kernel = functools.partial(
    pl.kernel,
    mesh=_sc_mesh,
    compiler_params=_sc_params,
    out_type=jax.ShapeDtypeStruct((N, F), jnp.float32),
    scratch_types=[
        pltpu.VMEM((DC,), jnp.int32),
        pltpu.VMEM((DC,), jnp.int32),
        pltpu.VMEM((MB_CAP,), jnp.int32),
        pltpu.VMEM((MB_CAP + L,), jnp.int32),
        pltpu.VMEM((MB_CAP, F), jnp.float32),
        pltpu.VMEM((PT, F), jnp.float32),
        pltpu.SemaphoreType.DMA((MB_CAP // 128,)),
        pltpu.SemaphoreType.DMA,
        pltpu.SemaphoreType.DMA,
    ],
)(_segmax_body)


def kernel(x, edge_index, x_label, W_label, W1, b1, W2, b2):
    src = edge_index[0]
    dst = edge_index[1]
    P = W_label[:, :F].T
    Q = W_label[:, F:].T
    w_uv = jnp.concatenate([(P - Q) @ W1.T, Q @ W1.T], axis=1)
    b_uv = jnp.concatenate([b1, jnp.zeros((F,), jnp.float32)])[None, :]

    u, v = pl.pallas_call(
        _uv_body,
        grid=(N // BN,),
        in_specs=[
            pl.BlockSpec((BN, F), lambda i: (i, 0)),
            pl.BlockSpec((F, 2 * F), lambda i: (0, 0)),
            pl.BlockSpec((1, 2 * F), lambda i: (0, 0)),
        ],
        out_specs=[
            pl.BlockSpec((BN, F), lambda i: (i, 0)),
            pl.BlockSpec((BN, F), lambda i: (i, 0)),
        ],
        out_shape=[
            jax.ShapeDtypeStruct((N, F), jnp.float32),
            jax.ShapeDtypeStruct((N, F), jnp.float32),
        ],
    )(x, w_uv, b_uv)

    g = _gather_kernel(u, v, dst, src)

    m = pl.pallas_call(
        _mlp_body,
        grid=(E // BE,),
        in_specs=[
            pl.BlockSpec((BE, F), lambda i: (i, 0)),
            pl.BlockSpec((BE, F), lambda i: (i, 0)),
            pl.BlockSpec((F, F), lambda i: (0, 0)),
            pl.BlockSpec((F, F), lambda i: (0, 0)),
            pl.BlockSpec((1, F), lambda i: (0, 0)),
        ],
        out_specs=pl.BlockSpec((BE, F), lambda i: (i, 0)),
        out_shape=jax.ShapeDtypeStruct((E, F), jnp.float32),
    )(x_label, g, W1.T, W2.T, b2[None, :])

    return _segmax_kernel(dst, m)
